# baseline (speedup 1.0000x reference)
import jax
import jax.numpy as jnp
from jax.experimental import pallas as pl
from jax.experimental.pallas import tpu as pltpu

N_TOKENS = 16384
D_IN = 4096
HIDDEN = 64
N_EXPERTS = 64
BT = 512


def _router_block(feat_ref, w1_ref, b1_ref, w2_ref, b2_ref, w3_ref, b3_ref,
                  hard_ref, probs_ref):
    f = feat_ref[...]
    h = jnp.dot(f, w1_ref[...], preferred_element_type=jnp.float32)
    h = jnp.maximum(h + b1_ref[...], 0.0)
    h = jnp.dot(h, w2_ref[...], preferred_element_type=jnp.float32)
    h = jnp.maximum(h + b2_ref[...], 0.0)
    logits = jnp.dot(h, w3_ref[...], preferred_element_type=jnp.float32)
    logits = logits + b3_ref[...]
    m = jnp.max(logits, axis=-1, keepdims=True)
    e = jnp.exp(logits - m)
    probs = e / jnp.sum(e, axis=-1, keepdims=True)
    probs_ref[...] = probs.T
    idx = jnp.argmax(probs, axis=-1)
    lane = jax.lax.broadcasted_iota(jnp.int32, probs.shape, 1)
    hard = jnp.where(lane == idx[:, None], 1.0, 0.0).astype(jnp.float32)
    hard_ref[...] = hard.T


@jax.jit
def kernel(feat, W1, b1, W2, b2, W3, b3):
    b1r = b1.reshape(1, HIDDEN)
    b2r = b2.reshape(1, HIDDEN)
    b3r = b3.reshape(1, N_EXPERTS)
    grid = (N_TOKENS // BT,)
    out = pl.pallas_call(
        _router_block,
        grid=grid,
        in_specs=[
            pl.BlockSpec((BT, D_IN), lambda i: (i, 0)),
            pl.BlockSpec((D_IN, HIDDEN), lambda i: (0, 0)),
            pl.BlockSpec((1, HIDDEN), lambda i: (0, 0)),
            pl.BlockSpec((HIDDEN, HIDDEN), lambda i: (0, 0)),
            pl.BlockSpec((1, HIDDEN), lambda i: (0, 0)),
            pl.BlockSpec((HIDDEN, N_EXPERTS), lambda i: (0, 0)),
            pl.BlockSpec((1, N_EXPERTS), lambda i: (0, 0)),
        ],
        out_specs=[
            pl.BlockSpec((N_EXPERTS, BT), lambda i: (0, i)),
            pl.BlockSpec((N_EXPERTS, BT), lambda i: (0, i)),
        ],
        out_shape=[
            jax.ShapeDtypeStruct((N_EXPERTS, N_TOKENS), jnp.float32),
            jax.ShapeDtypeStruct((N_EXPERTS, N_TOKENS), jnp.float32),
        ],
        compiler_params=pltpu.CompilerParams(
            dimension_semantics=("arbitrary",),
        ),
    )(feat, W1, b1r, W2, b2r, W3, b3r)
    return out[0].T, out[1].T


# speedup vs baseline: 1.1295x; 1.1295x over previous
import jax
import jax.numpy as jnp
from jax.experimental import pallas as pl
from jax.experimental.pallas import tpu as pltpu

N_TOKENS = 16384
D_IN = 4096
HIDDEN = 64
N_EXPERTS = 64
BT = 1024


def _router_block(feat_ref, w1_ref, b1_ref, w2_ref, b2_ref, w3_ref, b3_ref,
                  hard_ref, probs_ref):
    f = feat_ref[...]
    h = jax.lax.dot_general(f, w1_ref[...], (((1,), (1,)), ((), ())),
                            preferred_element_type=jnp.float32)
    h = jnp.maximum(h + b1_ref[...], 0.0)
    h = jnp.dot(h, w2_ref[...], preferred_element_type=jnp.float32)
    h = jnp.maximum(h + b2_ref[...], 0.0)
    logits = jnp.dot(h, w3_ref[...], preferred_element_type=jnp.float32)
    logits = logits + b3_ref[...]
    m = jnp.max(logits, axis=-1, keepdims=True)
    e = jnp.exp(logits - m)
    probs = e / jnp.sum(e, axis=-1, keepdims=True)
    probs_ref[...] = probs.T
    idx = jnp.argmax(probs, axis=-1)
    lane = jax.lax.broadcasted_iota(jnp.int32, probs.shape, 1)
    hard = jnp.where(lane == idx[:, None], 1.0, 0.0).astype(jnp.float32)
    hard_ref[...] = hard.T


@jax.jit
def kernel(feat, W1, b1, W2, b2, W3, b3):
    b1r = b1.reshape(1, HIDDEN)
    b2r = b2.reshape(1, HIDDEN)
    b3r = b3.reshape(1, N_EXPERTS)
    grid = (N_TOKENS // BT,)
    out = pl.pallas_call(
        _router_block,
        grid=grid,
        in_specs=[
            pl.BlockSpec((BT, D_IN), lambda i: (i, 0)),
            pl.BlockSpec((HIDDEN, D_IN), lambda i: (0, 0)),
            pl.BlockSpec((1, HIDDEN), lambda i: (0, 0)),
            pl.BlockSpec((HIDDEN, HIDDEN), lambda i: (0, 0)),
            pl.BlockSpec((1, HIDDEN), lambda i: (0, 0)),
            pl.BlockSpec((HIDDEN, N_EXPERTS), lambda i: (0, 0)),
            pl.BlockSpec((1, N_EXPERTS), lambda i: (0, 0)),
        ],
        out_specs=[
            pl.BlockSpec((N_EXPERTS, BT), lambda i: (0, i)),
            pl.BlockSpec((N_EXPERTS, BT), lambda i: (0, i)),
        ],
        out_shape=[
            jax.ShapeDtypeStruct((N_EXPERTS, N_TOKENS), jnp.float32),
            jax.ShapeDtypeStruct((N_EXPERTS, N_TOKENS), jnp.float32),
        ],
        compiler_params=pltpu.CompilerParams(
            dimension_semantics=("arbitrary",),
        ),
    )(feat, W1.T, b1r, W2, b2r, W3, b3r)
    return out[0].T, out[1].T
